# Initial kernel scaffold; baseline (speedup 1.0000x reference)
import functools

import jax
import jax.numpy as jnp
from jax import lax
from jax.experimental import pallas as pl
from jax.experimental.pallas import tpu as pltpu
from jax.experimental.pallas import tpu_sc as plsc

N = 10000
D = 256
E = 160000

NC = 2
NS = 16
H = D // NC
LC = 16
CH = 128
NCHUNKS = E // CH
KMAX = -(-NCHUNKS // NS)
RPT = N // NS

_mesh = plsc.VectorSubcoreMesh(core_axis_name="c", subcore_axis_name="s")


@functools.partial(
    pl.kernel,
    out_type=(
        jax.ShapeDtypeStruct((NC * N, H), jnp.float32),
        jax.ShapeDtypeStruct((N, LC), jnp.float32),
    ),
    mesh=_mesh,
    scratch_types=(
        pltpu.VMEM((CH,), jnp.int32),
        pltpu.VMEM((CH,), jnp.int32),
        pltpu.VMEM((CH, H), jnp.float32),
        pltpu.VMEM((CH, LC), jnp.float32),
        pltpu.VMEM_SHARED((N, H), jnp.float32),
        pltpu.VMEM_SHARED((N, LC), jnp.float32),
        pltpu.SemaphoreType.DMA,
    ),
)
def _sc_aggregate(x2, src2, dst, z_agg, z_cnt,
                  agg_out, cnt_out,
                  sidx, didx, rows, ones, agg_sh, cnt_sh, sem):
    c = lax.axis_index("c")
    s = lax.axis_index("s")

    r0 = s * RPT
    pltpu.sync_copy(z_agg.at[pl.ds(r0, RPT)], agg_sh.at[pl.ds(r0, RPT)])
    pltpu.sync_copy(z_cnt.at[pl.ds(r0, RPT)], cnt_sh.at[pl.ds(r0, RPT)])

    one_v = jnp.full((LC,), 1.0, dtype=jnp.float32)

    def _fill(i, _):
        ones[i] = one_v
        return 0

    lax.fori_loop(0, CH, _fill, 0)

    plsc.subcore_barrier()

    def _chunk(k, _):
        chunk = k * NS + s

        @pl.when(chunk < NCHUNKS)
        def _():
            off = pl.multiple_of(chunk * CH, CH)
            pltpu.sync_copy(src2.at[pl.ds(c * E + off, CH)], sidx)
            pltpu.sync_copy(dst.at[pl.ds(off, CH)], didx)
            pltpu.async_copy(x2.at[sidx], rows, sem).wait()
            pltpu.sync_copy(rows, agg_sh.at[didx], add=True)

            @pl.when(c == 0)
            def _():
                pltpu.sync_copy(ones, cnt_sh.at[didx], add=True)

        return 0

    lax.fori_loop(0, KMAX, _chunk, 0)

    plsc.subcore_barrier()

    pltpu.sync_copy(agg_sh.at[pl.ds(r0, RPT)],
                    agg_out.at[pl.ds(c * N + r0, RPT)])

    @pl.when(c == 0)
    def _():
        pltpu.sync_copy(cnt_sh.at[pl.ds(r0, RPT)], cnt_out.at[pl.ds(r0, RPT)])


BN = 1000


def _tc_body(agg0, agg1, cnt, x, wlt, wrt, b, o):
    r = 1.0 / jnp.maximum(cnt[:, 0:1], 1.0)
    mean = jnp.concatenate([agg0[...] * r, agg1[...] * r], axis=1)
    o[...] = (jnp.dot(mean, wlt[...], preferred_element_type=jnp.float32)
              + jnp.dot(x[...], wrt[...], preferred_element_type=jnp.float32)
              + b[...])


def _tc_dense(agg2, cnt, x, wlt, wrt, b):
    nb = N // BN
    return pl.pallas_call(
        _tc_body,
        grid=(nb,),
        in_specs=[
            pl.BlockSpec((BN, H), lambda i: (i, 0)),
            pl.BlockSpec((BN, H), lambda i: (i + nb, 0)),
            pl.BlockSpec((BN, LC), lambda i: (i, 0)),
            pl.BlockSpec((BN, D), lambda i: (i, 0)),
            pl.BlockSpec((D, D), lambda i: (0, 0)),
            pl.BlockSpec((D, D), lambda i: (0, 0)),
            pl.BlockSpec((1, D), lambda i: (0, 0)),
        ],
        out_specs=pl.BlockSpec((BN, D), lambda i: (i, 0)),
        out_shape=jax.ShapeDtypeStruct((N, D), jnp.float32),
    )(agg2, agg2, cnt, x, wlt, wrt, b)


def kernel(smashed_data, edge_index, W_l, b_l, W_r):
    x2 = jnp.transpose(smashed_data.reshape(N, NC, H), (1, 0, 2)).reshape(NC * N, H)
    src = edge_index[0]
    dst = edge_index[1]
    src2 = jnp.concatenate([src, src + N])
    z_agg = jnp.zeros((N, H), jnp.float32)
    z_cnt = jnp.zeros((N, LC), jnp.float32)

    agg2, cnt = _sc_aggregate(x2, src2, dst, z_agg, z_cnt)

    return _tc_dense(agg2, cnt, smashed_data,
                     W_l.T, W_r.T, b_l.reshape(1, D))


# speedup vs baseline: 2.8006x; 2.8006x over previous
import functools

import jax
import jax.numpy as jnp
from jax import lax
from jax.experimental import pallas as pl
from jax.experimental.pallas import tpu as pltpu
from jax.experimental.pallas import tpu_sc as plsc

N = 10000
D = 256
E = 160000

NC = 2
NS = 16
L = 16
H = D // NC
CH = 64
KE2 = -(-E // (CH * NS * NC))
KE = KE2 * NC
E2 = KE * CH * NS
EC = E2 // NC
BN = 1000
NP = 11264
KZ = NP // (CH * NS)

_mesh = plsc.VectorSubcoreMesh(core_axis_name="c", subcore_axis_name="s")


@functools.partial(
    pl.kernel,
    out_type=(
        jax.ShapeDtypeStruct((NC * NP, H), jnp.float32),
        jax.ShapeDtypeStruct((NC * NP, H), jnp.float32),
    ),
    mesh=_mesh,
    scratch_types=(
        pltpu.VMEM((CH,), jnp.int32),
        pltpu.VMEM((CH,), jnp.int32),
        pltpu.VMEM((CH,), jnp.int32),
        pltpu.VMEM((CH, H), jnp.float32),
        pltpu.VMEM_SHARED((NP, H), jnp.float32),
        pltpu.SemaphoreType.DMA,
    ),
)
def _sc_aggregate(x2, src2, dst2, z_blk, ones_in,
                  agg_out, cnt_out,
                  sidx, didx, zidx, rows, agg_sh, sem):
    c = lax.axis_index("c")
    s = lax.axis_index("s")

    lanes = lax.iota(jnp.int32, L)

    def _fill_zidx(base):
        for j in range(CH // L):
            zidx[pl.ds(j * L, L)] = base + j * L + lanes

    def _zero_accum():
        pltpu.sync_copy(z_blk, rows)

        @pl.loop(0, KZ)
        def _zero(k):
            _fill_zidx((k * NS + s) * CH)
            pltpu.sync_copy(rows, agg_sh.at[zidx])

    def _write_accum(out_ref):
        cbase = pl.multiple_of(c * NP, 8)

        @pl.loop(0, KZ)
        def _wout(k):
            base = pl.multiple_of((k * NS + s) * CH, CH)
            _fill_zidx(base)
            pltpu.async_copy(agg_sh.at[zidx], rows, sem).wait()
            pltpu.sync_copy(rows, out_ref.at[pl.ds(cbase + base, CH)])

    _zero_accum()
    plsc.subcore_barrier()

    @pl.loop(0, KE)
    def _chunk(k):
        off = pl.multiple_of((k * NS + s) * CH, CH)
        pltpu.sync_copy(src2.at[pl.ds(c * E2 + off, CH)], sidx)
        pltpu.sync_copy(dst2.at[pl.ds(off, CH)], didx)
        pltpu.async_copy(x2.at[sidx], rows, sem).wait()
        pltpu.sync_copy(rows, agg_sh.at[didx], add=True)

    plsc.subcore_barrier()
    _write_accum(agg_out)
    plsc.subcore_barrier()

    _zero_accum()
    plsc.subcore_barrier()
    pltpu.sync_copy(ones_in, rows)

    @pl.loop(0, KE2)
    def _cchunk(k):
        off = pl.multiple_of(c * EC + (k * NS + s) * CH, CH)
        pltpu.sync_copy(dst2.at[pl.ds(off, CH)], didx)
        pltpu.sync_copy(rows, agg_sh.at[didx], add=True)

    plsc.subcore_barrier()
    _write_accum(cnt_out)


def _tc_body(agg0, agg1, cnt0, cnt1, x, wlt, wrt, b, o):
    cnt = cnt0[:, 0:1] + cnt1[:, 0:1]
    r = 1.0 / jnp.maximum(cnt, 1.0)
    mean = jnp.concatenate([agg0[...] * r, agg1[...] * r], axis=1)
    o[...] = (jnp.dot(mean, wlt[...], preferred_element_type=jnp.float32)
              + jnp.dot(x[...], wrt[...], preferred_element_type=jnp.float32)
              + b[...])


def _tc_dense(agg0, agg1, cnt0, cnt1, x, wlt, wrt, b):
    nb = N // BN
    return pl.pallas_call(
        _tc_body,
        grid=(nb,),
        in_specs=[
            pl.BlockSpec((BN, H), lambda i: (i, 0)),
            pl.BlockSpec((BN, H), lambda i: (i, 0)),
            pl.BlockSpec((BN, H), lambda i: (i, 0)),
            pl.BlockSpec((BN, H), lambda i: (i, 0)),
            pl.BlockSpec((BN, D), lambda i: (i, 0)),
            pl.BlockSpec((D, D), lambda i: (0, 0)),
            pl.BlockSpec((D, D), lambda i: (0, 0)),
            pl.BlockSpec((1, D), lambda i: (0, 0)),
        ],
        out_specs=pl.BlockSpec((BN, D), lambda i: (i, 0)),
        out_shape=jax.ShapeDtypeStruct((N, D), jnp.float32),
    )(agg0, agg1, cnt0, cnt1, x, wlt, wrt, b)


def kernel(smashed_data, edge_index, W_l, b_l, W_r):
    x2 = jnp.transpose(smashed_data.reshape(N, NC, H), (1, 0, 2)).reshape(NC * N, H)
    src = edge_index[0]
    dst = edge_index[1]
    pad = E2 - E
    srcp = jnp.concatenate([src, jnp.zeros((pad,), jnp.int32)])
    src2 = jnp.concatenate([srcp, srcp + N])
    dst2 = jnp.concatenate([dst, jnp.full((pad,), N, jnp.int32)])
    z_blk = jnp.zeros((CH, H), jnp.float32)
    ones_in = jnp.ones((CH, H), jnp.float32)

    agg2, cntp = _sc_aggregate(x2, src2, dst2, z_blk, ones_in)
    agg0 = agg2[0:N]
    agg1 = agg2[NP:NP + N]
    cnt0 = cntp[0:N]
    cnt1 = cntp[NP:NP + N]

    return _tc_dense(agg0, agg1, cnt0, cnt1, smashed_data,
                     W_l.T, W_r.T, b_l.reshape(1, D))


# speedup vs baseline: 2.8536x; 1.0189x over previous
import functools

import jax
import jax.numpy as jnp
from jax import lax
from jax.experimental import pallas as pl
from jax.experimental.pallas import tpu as pltpu
from jax.experimental.pallas import tpu_sc as plsc

N = 10000
D = 256
E = 160000

NC = 2
NS = 16
L = 16
H = D // NC
CH = 64
KE2 = -(-E // (CH * NS * NC))
KE = KE2 * NC
E2 = KE * CH * NS
EC = E2 // NC
BN = 1000
NP = 11264
KZ = NP // (CH * NS)

_mesh = plsc.VectorSubcoreMesh(core_axis_name="c", subcore_axis_name="s")


@functools.partial(
    pl.kernel,
    out_type=(
        jax.ShapeDtypeStruct((NC * NP, H), jnp.float32),
        jax.ShapeDtypeStruct((NC * NP, H), jnp.float32),
    ),
    mesh=_mesh,
    scratch_types=(
        pltpu.VMEM((CH,), jnp.int32),
        pltpu.VMEM((CH,), jnp.int32),
        pltpu.VMEM((CH,), jnp.int32),
        pltpu.VMEM((CH, H), jnp.float32),
        pltpu.VMEM_SHARED((NP, H), jnp.float32),
        pltpu.SemaphoreType.DMA,
    ),
)
def _sc_aggregate(x2, src2, dst2, z_blk, ones_in,
                  agg_out, cnt_out,
                  sidx, didx, zidx, rows, agg_sh, sem):
    c = lax.axis_index("c")
    s = lax.axis_index("s")

    lanes = lax.iota(jnp.int32, L)

    def _fill_zidx(base):
        for j in range(CH // L):
            zidx[pl.ds(j * L, L)] = base + j * L + lanes

    def _zero_accum():
        pltpu.sync_copy(z_blk, rows)

        @pl.loop(0, KZ)
        def _zero(k):
            _fill_zidx((k * NS + s) * CH)
            pltpu.sync_copy(rows, agg_sh.at[zidx])

    def _write_accum(out_ref):
        cbase = pl.multiple_of(c * NP, 8)

        @pl.loop(0, KZ)
        def _wout(k):
            base = pl.multiple_of((k * NS + s) * CH, CH)
            _fill_zidx(base)
            pltpu.async_copy(agg_sh.at[zidx], rows, sem).wait()
            pltpu.sync_copy(rows, out_ref.at[pl.ds(cbase + base, CH)])

    _zero_accum()
    plsc.subcore_barrier()

    @pl.loop(0, KE)
    def _chunk(k):
        off = pl.multiple_of((k * NS + s) * CH, CH)
        pltpu.sync_copy(src2.at[pl.ds(c * E2 + off, CH)], sidx)
        pltpu.sync_copy(dst2.at[pl.ds(off, CH)], didx)
        pltpu.async_copy(x2.at[sidx], rows, sem).wait()
        pltpu.sync_copy(rows, agg_sh.at[didx], add=True)

    plsc.subcore_barrier()
    _write_accum(agg_out)
    plsc.subcore_barrier()

    _zero_accum()
    plsc.subcore_barrier()
    pltpu.sync_copy(ones_in, rows)

    @pl.loop(0, KE2)
    def _cchunk(k):
        off = pl.multiple_of(c * EC + (k * NS + s) * CH, CH)
        pltpu.sync_copy(dst2.at[pl.ds(off, CH)], didx)
        pltpu.sync_copy(rows, agg_sh.at[didx], add=True)

    plsc.subcore_barrier()
    _write_accum(cnt_out)


def _tc_body(agg0, agg1, cnt0, cnt1, x, wlt, wrt, b, o):
    cnt = cnt0[0][:, 0:1] + cnt1[0][:, 0:1]
    r = 1.0 / jnp.maximum(cnt, 1.0)
    mean = jnp.concatenate([agg0[0] * r, agg1[0] * r], axis=1)
    o[...] = (jnp.dot(mean, wlt[...], preferred_element_type=jnp.float32)
              + jnp.dot(x[...], wrt[...], preferred_element_type=jnp.float32)
              + b[...])


def _tc_dense(agg2, cntp, x, wlt, wrt, b):
    nb = N // BN
    return pl.pallas_call(
        _tc_body,
        grid=(nb,),
        in_specs=[
            pl.BlockSpec((1, BN, H), lambda i: (0, i, 0)),
            pl.BlockSpec((1, BN, H), lambda i: (1, i, 0)),
            pl.BlockSpec((1, BN, H), lambda i: (0, i, 0)),
            pl.BlockSpec((1, BN, H), lambda i: (1, i, 0)),
            pl.BlockSpec((BN, D), lambda i: (i, 0)),
            pl.BlockSpec((D, D), lambda i: (0, 0)),
            pl.BlockSpec((D, D), lambda i: (0, 0)),
            pl.BlockSpec((1, D), lambda i: (0, 0)),
        ],
        out_specs=pl.BlockSpec((BN, D), lambda i: (i, 0)),
        out_shape=jax.ShapeDtypeStruct((N, D), jnp.float32),
    )(agg2, agg2, cntp, cntp, x, wlt, wrt, b)


def kernel(smashed_data, edge_index, W_l, b_l, W_r):
    x2 = jnp.transpose(smashed_data.reshape(N, NC, H), (1, 0, 2)).reshape(NC * N, H)
    src = edge_index[0]
    dst = edge_index[1]
    pad = E2 - E
    srcp = jnp.concatenate([src, jnp.zeros((pad,), jnp.int32)])
    src2 = jnp.concatenate([srcp, srcp + N])
    dst2 = jnp.concatenate([dst, jnp.full((pad,), N, jnp.int32)])
    z_blk = jnp.zeros((CH, H), jnp.float32)
    ones_in = jnp.ones((CH, H), jnp.float32)

    agg2, cntp = _sc_aggregate(x2, src2, dst2, z_blk, ones_in)
    agg2r = agg2.reshape(NC, NP, H)
    cntr = cntp.reshape(NC, NP, H)

    return _tc_dense(agg2r, cntr, smashed_data,
                     W_l.T, W_r.T, b_l.reshape(1, D))


# speedup vs baseline: 3.1325x; 1.0977x over previous
import functools

import jax
import jax.numpy as jnp
from jax import lax
from jax.experimental import pallas as pl
from jax.experimental.pallas import tpu as pltpu
from jax.experimental.pallas import tpu_sc as plsc

N = 10000
D = 256
E = 160000

NC = 2
NS = 16
L = 16
H = D // NC
CH = 128
KE2 = -(-E // (CH * NS * NC))
KE = KE2 * NC
E2 = KE * CH * NS
EC = E2 // NC
BN = 1000
NP = 12288
KZ = NP // (CH * NS)

_mesh = plsc.VectorSubcoreMesh(core_axis_name="c", subcore_axis_name="s")


@functools.partial(
    pl.kernel,
    out_type=(
        jax.ShapeDtypeStruct((NC * NP, H), jnp.float32),
        jax.ShapeDtypeStruct((NC * NP, H), jnp.float32),
    ),
    mesh=_mesh,
    scratch_types=(
        pltpu.VMEM((CH,), jnp.int32),
        pltpu.VMEM((CH,), jnp.int32),
        pltpu.VMEM((CH,), jnp.int32),
        pltpu.VMEM((CH, H), jnp.float32),
        pltpu.VMEM_SHARED((NP, H), jnp.float32),
        pltpu.SemaphoreType.DMA,
    ),
)
def _sc_aggregate(x2, src2, dst2, z_blk, ones_in,
                  agg_out, cnt_out,
                  sidx, didx, zidx, rows, agg_sh, sem):
    c = lax.axis_index("c")
    s = lax.axis_index("s")

    lanes = lax.iota(jnp.int32, L)

    def _fill_zidx(base):
        for j in range(CH // L):
            zidx[pl.ds(j * L, L)] = base + j * L + lanes

    def _zero_accum():
        pltpu.sync_copy(z_blk, rows)

        @pl.loop(0, KZ)
        def _zero(k):
            _fill_zidx((k * NS + s) * CH)
            pltpu.sync_copy(rows, agg_sh.at[zidx])

    def _write_accum(out_ref):
        cbase = pl.multiple_of(c * NP, 8)

        @pl.loop(0, KZ)
        def _wout(k):
            base = pl.multiple_of((k * NS + s) * CH, CH)
            _fill_zidx(base)
            pltpu.async_copy(agg_sh.at[zidx], rows, sem).wait()
            pltpu.sync_copy(rows, out_ref.at[pl.ds(cbase + base, CH)])

    _zero_accum()
    plsc.subcore_barrier()

    @pl.loop(0, KE)
    def _chunk(k):
        off = pl.multiple_of((k * NS + s) * CH, CH)
        pltpu.sync_copy(src2.at[pl.ds(c * E2 + off, CH)], sidx)
        pltpu.sync_copy(dst2.at[pl.ds(off, CH)], didx)
        pltpu.async_copy(x2.at[sidx], rows, sem).wait()
        pltpu.sync_copy(rows, agg_sh.at[didx], add=True)

    plsc.subcore_barrier()
    _write_accum(agg_out)
    plsc.subcore_barrier()

    _zero_accum()
    plsc.subcore_barrier()
    pltpu.sync_copy(ones_in, rows)

    @pl.loop(0, KE2)
    def _cchunk(k):
        off = pl.multiple_of(c * EC + (k * NS + s) * CH, CH)
        pltpu.sync_copy(dst2.at[pl.ds(off, CH)], didx)
        pltpu.sync_copy(rows, agg_sh.at[didx], add=True)

    plsc.subcore_barrier()
    _write_accum(cnt_out)


def _tc_body(agg0, agg1, cnt0, cnt1, x, wlt, wrt, b, o):
    cnt = cnt0[0][:, 0:1] + cnt1[0][:, 0:1]
    r = 1.0 / jnp.maximum(cnt, 1.0)
    mean = jnp.concatenate([agg0[0] * r, agg1[0] * r], axis=1)
    o[...] = (jnp.dot(mean, wlt[...], preferred_element_type=jnp.float32)
              + jnp.dot(x[...], wrt[...], preferred_element_type=jnp.float32)
              + b[...])


def _tc_dense(agg2, cntp, x, wlt, wrt, b):
    nb = N // BN
    return pl.pallas_call(
        _tc_body,
        grid=(nb,),
        in_specs=[
            pl.BlockSpec((1, BN, H), lambda i: (0, i, 0)),
            pl.BlockSpec((1, BN, H), lambda i: (1, i, 0)),
            pl.BlockSpec((1, BN, H), lambda i: (0, i, 0)),
            pl.BlockSpec((1, BN, H), lambda i: (1, i, 0)),
            pl.BlockSpec((BN, D), lambda i: (i, 0)),
            pl.BlockSpec((D, D), lambda i: (0, 0)),
            pl.BlockSpec((D, D), lambda i: (0, 0)),
            pl.BlockSpec((1, D), lambda i: (0, 0)),
        ],
        out_specs=pl.BlockSpec((BN, D), lambda i: (i, 0)),
        out_shape=jax.ShapeDtypeStruct((N, D), jnp.float32),
    )(agg2, agg2, cntp, cntp, x, wlt, wrt, b)


def kernel(smashed_data, edge_index, W_l, b_l, W_r):
    x2 = jnp.transpose(smashed_data.reshape(N, NC, H), (1, 0, 2)).reshape(NC * N, H)
    src = edge_index[0]
    dst = edge_index[1]
    pad = E2 - E
    srcp = jnp.concatenate([src, jnp.zeros((pad,), jnp.int32)])
    src2 = jnp.concatenate([srcp, srcp + N])
    dst2 = jnp.concatenate([dst, jnp.full((pad,), N, jnp.int32)])
    z_blk = jnp.zeros((CH, H), jnp.float32)
    ones_in = jnp.ones((CH, H), jnp.float32)

    agg2, cntp = _sc_aggregate(x2, src2, dst2, z_blk, ones_in)
    agg2r = agg2.reshape(NC, NP, H)
    cntr = cntp.reshape(NC, NP, H)

    return _tc_dense(agg2r, cntr, smashed_data,
                     W_l.T, W_r.T, b_l.reshape(1, D))
